# baseline (speedup 1.0000x reference)
import functools

import numpy as np
import jax
import jax.numpy as jnp
from jax import lax
from jax.experimental import pallas as pl
from jax.experimental.pallas import tpu as pltpu
from jax.experimental.pallas import tpu_sc as plsc

_VOCAB = 1000000
_D = 64
_MAX_LEN = 256
_B, _L = 4096, 200
_NW = 32
_BW = _B // _NW
_NBUF = 4
_NOB = 2
_NSTEP = _L // _NBUF


def _sinusoidal_pe(max_len, d):
    position = np.arange(max_len, dtype=np.float32)[:, None]
    div_term = np.exp(
        np.arange(0, d, 2, dtype=np.float32) * (-np.log(10000.0) / d))
    pe = np.zeros((max_len, d), dtype=np.float32)
    pe[:, 0::2] = np.sin(position * div_term)
    pe[:, 1::2] = np.cos(position * div_term)
    return pe


_PE2 = jnp.asarray(_sinusoidal_pe(_MAX_LEN, _D)[:_L].reshape(_L // 2, 2 * _D))


def _emb_body(tokens_t_hbm, table2_hbm, pe_hbm, out_hbm,
              idx_v, jdx_v, rows_v, obuf_v, pe_v, gsem, osem):
    w = lax.axis_index("s") * 2 + lax.axis_index("c")
    b0 = w * _BW

    pltpu.sync_copy(tokens_t_hbm.at[:, pl.ds(b0, _BW)], idx_v)
    pltpu.sync_copy(pe_hbm, pe_v)

    lanes = lax.iota(jnp.int32, 16)

    def start_gather(l, b):
        for g in range(_BW // 16):
            tok16 = idx_v[l, pl.ds(g * 16, 16)]
            jdx_v[b, pl.ds(g * 16, 16)] = lax.shift_right_logical(tok16, 1)
        pltpu.async_copy(
            table2_hbm.at[jdx_v.at[b]], rows_v.at[b], gsem.at[b])

    def wait_gather(b):
        pltpu.make_async_copy(
            table2_hbm.at[pl.ds(0, _BW), :], rows_v.at[b], gsem.at[b]).wait()

    def wait_out(o):
        pltpu.make_async_copy(
            obuf_v.at[o], out_hbm.at[0, :, pl.ds(0, _BW)], osem.at[o]).wait()

    def compute(l, b, o):
        prow = l // 2
        pcol = lax.rem(l, 2) * _D
        bvec = jnp.full((16,), b, jnp.int32)
        gprep = []
        for g in range(_BW // 16):
            tok16 = idx_v[l, pl.ds(g * 16, 16)]
            m16 = jnp.where(tok16 == 0, 0.0, 1.0).astype(jnp.float32)
            col64 = (tok16 & 1) * _D
            gprep.append((lanes + g * 16, m16, col64))

        def d_body(d, carry):
            ped = plsc.load_gather(
                pe_v, [jnp.full((16,), prow, jnp.int32),
                       jnp.full((16,), pcol + d, jnp.int32)])
            for g in range(_BW // 16):
                jv, m16, col64 = gprep[g]
                val = plsc.load_gather(rows_v, [bvec, jv, col64 + d])
                obuf_v[o, d, pl.ds(g * 16, 16)] = val * m16 + ped
            return carry

        lax.fori_loop(0, _D, d_body, 0)

    start_gather(0, 0)
    start_gather(1, 1)

    def step_body(step, carry):
        for b in range(_NBUF):
            l = step * _NBUF + b
            o = b % _NOB
            b2 = (b + 2) % _NBUF

            if b < 2:
                start_gather(l + 2, b2)
            else:
                @pl.when(step < _NSTEP - 1)
                def _ahead():
                    start_gather(l + 2, b2)

            if b < 2:
                @pl.when(step > 0)
                def _drain():
                    wait_out(o)
            else:
                wait_out(o)

            wait_gather(b)
            compute(l, b, o)
            pltpu.async_copy(
                obuf_v.at[o], out_hbm.at[l, :, pl.ds(b0, _BW)], osem.at[o])
        return carry

    lax.fori_loop(0, _NSTEP, step_body, 0)

    wait_out(0)
    wait_out(1)


@jax.jit
def _emb(tokens_t, table2, pe):
    mesh = plsc.VectorSubcoreMesh(core_axis_name="c", subcore_axis_name="s")
    run = functools.partial(
        pl.kernel,
        out_type=jax.ShapeDtypeStruct((_L, _D, _B), jnp.float32),
        mesh=mesh,
        scratch_types=[
            pltpu.VMEM((_L, _BW), jnp.int32),
            pltpu.VMEM((_NBUF, _BW), jnp.int32),
            pltpu.VMEM((_NBUF, _BW, 2 * _D), jnp.float32),
            pltpu.VMEM((_NOB, _D, _BW), jnp.float32),
            pltpu.VMEM((_L // 2, 2 * _D), jnp.float32),
            pltpu.SemaphoreType.DMA((_NBUF,)),
            pltpu.SemaphoreType.DMA((_NOB,)),
        ],
        compiler_params=pltpu.CompilerParams(
            use_tc_tiling_on_sc=True, needs_layout_passes=False),
    )(_emb_body)
    return run(tokens_t, table2, pe)


def kernel(tokens, table):
    out_ldb = _emb(tokens.astype(jnp.int32).T,
                   table.reshape(_VOCAB // 2, 2 * _D), _PE2)
    return out_ldb.transpose(2, 0, 1)


# speedup vs baseline: 1.7117x; 1.7117x over previous
import functools

import numpy as np
import jax
import jax.numpy as jnp
from jax import lax
from jax.experimental import pallas as pl
from jax.experimental.pallas import tpu as pltpu
from jax.experimental.pallas import tpu_sc as plsc

_VOCAB = 1000000
_D = 64
_MAX_LEN = 256
_B, _L = 4096, 200
_NW = 32
_BW = _B // _NW
_NBUF = 4
_NSTEP = _L // _NBUF


def _sinusoidal_pe(max_len, d):
    position = np.arange(max_len, dtype=np.float32)[:, None]
    div_term = np.exp(
        np.arange(0, d, 2, dtype=np.float32) * (-np.log(10000.0) / d))
    pe = np.zeros((max_len, d), dtype=np.float32)
    pe[:, 0::2] = np.sin(position * div_term)
    pe[:, 1::2] = np.cos(position * div_term)
    return pe


_PE = jnp.asarray(_sinusoidal_pe(_MAX_LEN, _D)[:_L])

_BCAST_DNUMS = lax.GatherDimensionNumbers(
    offset_dims=(), collapsed_slice_dims=(0,), start_index_map=(0,))


def _bcast_lane(vec16, r):
    idx = jnp.full((16, 1), r, jnp.int32)
    return lax.gather(vec16, idx, _BCAST_DNUMS, slice_sizes=(1,),
                      mode=lax.GatherScatterMode.PROMISE_IN_BOUNDS)


def _emb_body(tokens_t_hbm, table_hbm, pe_hbm, out_hbm,
              idx_v, rows_v, pe_v, gsem, osem):
    w = lax.axis_index("s") * 2 + lax.axis_index("c")
    b0 = w * _BW

    pltpu.sync_copy(tokens_t_hbm.at[:, pl.ds(b0, _BW)], idx_v)
    pltpu.sync_copy(pe_hbm, pe_v)

    def start_gather(l, b):
        pltpu.async_copy(
            table_hbm.at[idx_v.at[l, :]], rows_v.at[b], gsem.at[b])

    def wait_gather(b):
        pltpu.make_async_copy(
            table_hbm.at[pl.ds(0, _BW), :], rows_v.at[b], gsem.at[b]).wait()

    def wait_out(b):
        pltpu.make_async_copy(
            rows_v.at[b], out_hbm.at[0, pl.ds(0, _BW), :], osem.at[b]).wait()

    def compute(l, b):
        pe_r = [pe_v[l, pl.ds(d * 16, 16)] for d in range(_D // 16)]

        def grp_body(g, gcarry):
            tok16 = idx_v[l, pl.ds(g * 16, 16)]
            m16 = jnp.where(tok16 == 0, 0.0, 1.0).astype(jnp.float32)
            for r in range(16):
                mb = _bcast_lane(m16, r)
                j = g * 16 + r
                for d in range(_D // 16):
                    sl = pl.ds(d * 16, 16)
                    rows_v[b, j, sl] = rows_v[b, j, sl] * mb + pe_r[d]
            return gcarry

        lax.fori_loop(0, _BW // 16, grp_body, 0)

    start_gather(0, 0)
    start_gather(1, 1)

    def step_body(step, carry):
        for b in range(_NBUF):
            l = step * _NBUF + b
            b2 = (b + 2) % _NBUF

            if b < 2:
                @pl.when(step > 0)
                def _recycle():
                    wait_out(b2)
            else:
                wait_out(b2)
            if b < 2:
                start_gather(l + 2, b2)
            else:
                @pl.when(step < _NSTEP - 1)
                def _ahead():
                    start_gather(l + 2, b2)

            wait_gather(b)
            compute(l, b)
            pltpu.async_copy(
                rows_v.at[b], out_hbm.at[l, pl.ds(b0, _BW), :], osem.at[b])
        return carry

    lax.fori_loop(0, _NSTEP, step_body, 0)

    wait_out(2)
    wait_out(3)


@jax.jit
def _emb(tokens_t, table, pe):
    mesh = plsc.VectorSubcoreMesh(core_axis_name="c", subcore_axis_name="s")
    run = functools.partial(
        pl.kernel,
        out_type=jax.ShapeDtypeStruct((_L, _B, _D), jnp.float32),
        mesh=mesh,
        scratch_types=[
            pltpu.VMEM((_L, _BW), jnp.int32),
            pltpu.VMEM((_NBUF, _BW, _D), jnp.float32),
            pltpu.VMEM((_L, _D), jnp.float32),
            pltpu.SemaphoreType.DMA((_NBUF,)),
            pltpu.SemaphoreType.DMA((_NBUF,)),
        ],
        compiler_params=pltpu.CompilerParams(use_tc_tiling_on_sc=False),
    )(_emb_body)
    return run(tokens_t, table, pe)


def kernel(tokens, table):
    out_t = _emb(tokens.astype(jnp.int32).T, table, _PE)
    return out_t.transpose(1, 0, 2)
